# Initial kernel scaffold; baseline (speedup 1.0000x reference)
import jax
import jax.numpy as jnp
from jax.experimental import pallas as pl


def kernel(features_omics1, features_omics1_a, features_omics2, features_omics2_a, edge_index_omics1, edge_weight_omics1, edge_index_omics2, edge_weight_omics2, W1_omics1, W2_omics1, Wdisc_omics1, bdisc_omics1, W1_omics2, W2_omics2, Wdisc_omics2, bdisc_omics2, Wc, bc, Wdec_omics1, Wdec_omics2):
    raise NotImplementedError("write your pallas kernel here")


# speedup vs baseline: 1.8003x; 1.8003x over previous
import functools

import jax
import jax.numpy as jnp
from jax import lax
from jax.experimental import pallas as pl
from jax.experimental.pallas import tpu as pltpu
from jax.experimental.pallas import tpu_sc as plsc

_W = 128
_CH = 128


def _sc_spmm(table, src, dst, w, n, *, relu, edge_split):
    epad = src.shape[0]
    per_tile = epad // (32 if edge_split else 16)
    n_chunks = per_tile // _CH
    rows_lo = (n // 16 // 8) * 8
    rows_hi = n - 15 * rows_lo

    mesh = plsc.VectorSubcoreMesh(core_axis_name="c", subcore_axis_name="s")

    @functools.partial(
        pl.kernel,
        mesh=mesh,
        compiler_params=pltpu.CompilerParams(needs_layout_passes=False),
        out_type=jax.ShapeDtypeStruct((2 * n, _W), jnp.float32),
        scratch_types=[
            pltpu.VMEM((_CH,), jnp.int32),
            pltpu.VMEM((_CH,), jnp.int32),
            pltpu.VMEM((_CH,), jnp.float32),
            pltpu.VMEM((_CH, _W), jnp.float32),
            pltpu.VMEM_SHARED((n, _W), jnp.float32),
            pltpu.SemaphoreType.DMA,
        ],
    )
    def _k(table_h, src_h, dst_h, w_h, out_h, src_v, dst_v, w_v, rows_v, acc, sem):
        c = lax.axis_index("c")
        s = lax.axis_index("s")

        def _zrow(i, carry):
            for j in range(_W // 16):
                rows_v[i, pl.ds(j * 16, 16)] = jnp.zeros((16,), jnp.float32)
            return carry
        lax.fori_loop(0, _CH, _zrow, 0)
        r0 = s * rows_lo
        full_lo, rem_lo = divmod(rows_lo, _CH)
        full_hi, rem_hi = divmod(rows_hi, _CH)

        def _zero_span(row0, count_full, remainder):
            for k in range(count_full):
                pltpu.sync_copy(rows_v, acc.at[pl.ds(row0 + k * _CH, _CH)])
            if remainder:
                pltpu.sync_copy(rows_v.at[pl.ds(0, remainder)],
                                acc.at[pl.ds(row0 + count_full * _CH,
                                             remainder)])

        @pl.when(s < 15)
        def _():
            _zero_span(r0, full_lo, rem_lo)

        @pl.when(s == 15)
        def _():
            _zero_span(r0, full_hi, rem_hi)
        plsc.subcore_barrier()

        base = (((c * 16 + s) if edge_split else s)) * per_tile

        def _chunk(t, carry):
            e0 = base + t * _CH
            pltpu.sync_copy(src_h.at[pl.ds(e0, _CH)], src_v)
            pltpu.sync_copy(dst_h.at[pl.ds(e0, _CH)], dst_v)
            pltpu.sync_copy(w_h.at[pl.ds(e0, _CH)], w_v)
            if not edge_split:
                off = c * n
                for j in range(_CH // 16):
                    src_v[pl.ds(j * 16, 16)] = src_v[pl.ds(j * 16, 16)] + off
            pltpu.async_copy(table_h.at[src_v], rows_v, sem).wait()

            def _row(i, rcarry):
                wv = plsc.load_gather(w_v, [jnp.zeros((16,), jnp.int32) + i])
                for j in range(_W // 16):
                    x = rows_v[i, pl.ds(j * 16, 16)]
                    if relu:
                        x = jnp.maximum(x, 0.0)
                    rows_v[i, pl.ds(j * 16, 16)] = x * wv
                return rcarry
            lax.fori_loop(0, _CH, _row, 0)

            pltpu.sync_copy(rows_v, acc.at[dst_v], add=True)
            return carry
        lax.fori_loop(0, n_chunks, _chunk, 0)
        plsc.subcore_barrier()

        out_base = c * n + r0

        @pl.when(s < 15)
        def _():
            pltpu.sync_copy(acc.at[pl.ds(r0, rows_lo)],
                            out_h.at[pl.ds(out_base, rows_lo)])

        @pl.when(s == 15)
        def _():
            pltpu.sync_copy(acc.at[pl.ds(r0, rows_hi)],
                            out_h.at[pl.ds(out_base, rows_hi)])

    return _k(table, src, dst, w)


def _tc_matmul(x, wmat, bm):
    m, kdim = x.shape
    kout = wmat.shape[1]

    def body(x_ref, w_ref, o_ref):
        o_ref[...] = jnp.dot(x_ref[...], w_ref[...],
                             preferred_element_type=jnp.float32)

    return pl.pallas_call(
        body,
        grid=(m // bm,),
        in_specs=[pl.BlockSpec((bm, kdim), lambda i: (i, 0)),
                  pl.BlockSpec((kdim, kout), lambda i: (0, 0))],
        out_specs=pl.BlockSpec((bm, kout), lambda i: (i, 0)),
        out_shape=jax.ShapeDtypeStruct((m, kout), jnp.float32),
    )(x, wmat)


def _tc_epilogue(z1s, v1s, z2s, v2s, wd1, wd2, wc, bc2d, bd2d, wdec1, wdec2):
    n = z1s.shape[1]
    bm = 1000

    def body(z1_ref, v1_ref, z2_ref, v2_ref, wd1_ref, wd2_ref, wc_ref,
             bc_ref, bd_ref, wdec1_ref, wdec2_ref,
             comb_ref, rets_ref, d1_ref, d2_ref):
        def gfun(v):
            nrm = jnp.sqrt(jnp.sum(v * v, axis=1, keepdims=True))
            return jax.nn.sigmoid(v / jnp.maximum(nrm, 1e-12))

        def mm(a, b):
            return jnp.dot(a, b, preferred_element_type=jnp.float32)

        def rd(a, b):
            return jnp.sum(a * b, axis=1, keepdims=True)

        e1 = jnp.maximum(z1_ref[0], 0.0)
        e1a = jnp.maximum(z1_ref[1], 0.0)
        e2 = jnp.maximum(z2_ref[0], 0.0)
        e2a = jnp.maximum(z2_ref[1], 0.0)
        g1 = gfun(v1_ref[0])
        g1a = gfun(v1_ref[1])
        g2 = gfun(v2_ref[0])
        g2a = gfun(v2_ref[1])
        p1 = mm(e1, wd1_ref[...])
        p1a = mm(e1a, wd1_ref[...])
        p2 = mm(e2, wd2_ref[...])
        p2a = mm(e2a, wd2_ref[...])
        rets = jnp.concatenate(
            [rd(p1, g1), rd(p1a, g1), rd(p1a, g1a), rd(p1, g1a),
             rd(p2, g2), rd(p2a, g2), rd(p2a, g2a), rd(p2, g2a)], axis=1)
        rets_ref[...] = rets + bd_ref[...]
        comb = mm(e1, wc_ref[0:128]) + mm(e2, wc_ref[128:256]) + bc_ref[...]
        comb = jnp.maximum(comb, 0.0)
        comb_ref[...] = comb
        d1_ref[0] = mm(comb, wdec1_ref[:, 0:128])
        d1_ref[1] = mm(comb, wdec1_ref[:, 128:256])
        d2_ref[...] = mm(comb, wdec2_ref[...])

    stk = lambda i: (0, i, 0)
    blk = lambda i: (i, 0)
    fix = lambda i: (0, 0)
    return pl.pallas_call(
        body,
        grid=(n // bm,),
        in_specs=[
            pl.BlockSpec((2, bm, 128), stk),
            pl.BlockSpec((2, bm, 128), stk),
            pl.BlockSpec((2, bm, 128), stk),
            pl.BlockSpec((2, bm, 128), stk),
            pl.BlockSpec((128, 128), fix),
            pl.BlockSpec((128, 128), fix),
            pl.BlockSpec((256, 128), fix),
            pl.BlockSpec((1, 128), fix),
            pl.BlockSpec((1, 8), fix),
            pl.BlockSpec((128, 256), fix),
            pl.BlockSpec((128, 128), fix),
        ],
        out_specs=[
            pl.BlockSpec((bm, 128), blk),
            pl.BlockSpec((bm, 8), blk),
            pl.BlockSpec((2, bm, 128), stk),
            pl.BlockSpec((bm, 128), blk),
        ],
        out_shape=[
            jax.ShapeDtypeStruct((n, 128), jnp.float32),
            jax.ShapeDtypeStruct((n, 8), jnp.float32),
            jax.ShapeDtypeStruct((2, n, 128), jnp.float32),
            jax.ShapeDtypeStruct((n, 128), jnp.float32),
        ],
    )(z1s, v1s, z2s, v2s, wd1, wd2, wc, bc2d, bd2d, wdec1, wdec2)


def _tc_pairsum(x2):
    n = x2.shape[1]
    bm = 2000

    def body(x_ref, o_ref):
        o_ref[...] = x_ref[0] + x_ref[1]

    return pl.pallas_call(
        body,
        grid=(n // bm,),
        in_specs=[pl.BlockSpec((2, bm, 128), lambda i: (0, i, 0))],
        out_specs=pl.BlockSpec((bm, 128), lambda i: (i, 0)),
        out_shape=jax.ShapeDtypeStruct((n, 128), jnp.float32),
    )(x2)


def kernel(features_omics1, features_omics1_a, features_omics2,
           features_omics2_a, edge_index_omics1, edge_weight_omics1,
           edge_index_omics2, edge_weight_omics2,
           W1_omics1, W2_omics1, Wdisc_omics1, bdisc_omics1,
           W1_omics2, W2_omics2, Wdisc_omics2, bdisc_omics2,
           Wc, bc, Wdec_omics1, Wdec_omics2):
    n = features_omics1.shape[0]
    e = edge_index_omics1.shape[1]
    epad = -(-e // 4096) * 4096

    def prep(ei, ew):
        pad = epad - e
        src = jnp.concatenate([ei[0].astype(jnp.int32),
                               jnp.zeros((pad,), jnp.int32)])
        dstv = jnp.concatenate([ei[1].astype(jnp.int32),
                                jnp.zeros((pad,), jnp.int32)])
        wv = jnp.concatenate([ew, jnp.zeros((pad,), jnp.float32)])
        return src, dstv, wv

    s1, t1, w1 = prep(edge_index_omics1, edge_weight_omics1)
    s2, t2, w2 = prep(edge_index_omics2, edge_weight_omics2)

    x1 = _tc_matmul(jnp.concatenate([features_omics1, features_omics1_a], 0),
                    W1_omics1, 2000)
    x2 = _tc_matmul(jnp.concatenate([features_omics2, features_omics2_a], 0),
                    W1_omics2, 2000)

    z1s = _sc_spmm(x1, s1, t1, w1, n, relu=False, edge_split=False)
    z2s = _sc_spmm(x2, s2, t2, w2, n, relu=False, edge_split=False)
    v1s = _sc_spmm(z1s, s1, t1, w1, n, relu=True, edge_split=False)
    v2s = _sc_spmm(z2s, s2, t2, w2, n, relu=True, edge_split=False)

    bd2d = jnp.concatenate([jnp.broadcast_to(bdisc_omics1, (4,)),
                            jnp.broadcast_to(bdisc_omics2, (4,))]).reshape(1, 8)
    comb, rets, d1t, d2t = _tc_epilogue(
        z1s.reshape(2, n, 128), v1s.reshape(2, n, 128),
        z2s.reshape(2, n, 128), v2s.reshape(2, n, 128),
        Wdisc_omics1[0], Wdisc_omics2[0], Wc, bc.reshape(1, 128), bd2d,
        Wdec_omics1, Wdec_omics2)

    r1s = _sc_spmm(d1t.reshape(2 * n, 128), s1, t1, w1, n,
                   relu=False, edge_split=False)
    r2p = _sc_spmm(d2t, s2, t2, w2, n, relu=False, edge_split=True)

    recon1 = jnp.concatenate([r1s[:n], r1s[n:]], axis=1)
    recon2 = _tc_pairsum(r2p.reshape(2, n, 128))

    h1 = z1s[:n]
    h2 = z2s[:n]
    ret1 = rets[:, 0:2]
    ret1_a = rets[:, 2:4]
    ret2 = rets[:, 4:6]
    ret2_a = rets[:, 6:8]
    return (h1, h2, comb, recon1, recon2, ret1, ret1_a, ret2, ret2_a)
